# baseline (speedup 1.0000x reference)
import dataclasses
import functools

import jax
import jax.numpy as jnp
from jax import lax
from jax.experimental import pallas as pl
from jax.experimental.pallas import tpu as pltpu
from jax.experimental.pallas import tpu_sc as plsc

_LANES = 16
_NWORKERS = 32
_IDX_CHUNK = 128


def _interp_kernel(B, L, C, T):
    bpw = B // _NWORKERS
    nchunk = (2 * T) // _IDX_CHUNK
    groups = T // _LANES
    mesh = plsc.VectorSubcoreMesh(core_axis_name="c", subcore_axis_name="s")
    cp = pltpu.CompilerParams()
    if "needs_layout_passes" in pltpu.CompilerParams.__dataclass_fields__:
        cp = dataclasses.replace(cp, needs_layout_passes=False)
    if "use_tc_tiling_on_sc" in pltpu.CompilerParams.__dataclass_fields__:
        cp = dataclasses.replace(cp, use_tc_tiling_on_sc=False)

    @functools.partial(
        pl.kernel,
        out_type=jax.ShapeDtypeStruct((B, T, C), jnp.float32),
        mesh=mesh,
        compiler_params=cp,
        scratch_types=[
            pltpu.VMEM((T,), jnp.float32),
            pltpu.VMEM((nchunk, _IDX_CHUNK), jnp.int32),
            pltpu.VMEM((nchunk, _IDX_CHUNK), jnp.int32),
            pltpu.VMEM((T, _LANES), jnp.float32),
            pltpu.VMEM((2 * T, C), jnp.float32),
            pltpu.VMEM((T, C), jnp.float32),
            pltpu.SemaphoreType.DMA,
        ],
    )
    def k(cf_hbm, tq_hbm, out_hbm, tq_v, bidx, idxb, fracx, rows, out_v, gsem):
        wid = lax.axis_index("s") * 2 + lax.axis_index("c")
        pltpu.sync_copy(tq_hbm, tq_v)

        lane = jax.lax.broadcasted_iota(jnp.int32, (_LANES,), 0)

        @pl.loop(0, groups)
        def _(g):
            t = tq_v[pl.ds(g * _LANES, _LANES)]
            ti = jnp.minimum(jnp.maximum(t.astype(jnp.int32), 0), L - 2)
            fr = t - ti.astype(jnp.float32)
            row = jnp.full((_LANES,), g // 4, jnp.int32)
            colbase = (g % 4) * 32 + 2 * lane
            plsc.store_scatter(bidx, [row, colbase], ti)
            plsc.store_scatter(bidx, [row, colbase + 1], ti + 1)
            j = g * _LANES + lane
            for c in range(_LANES):
                plsc.store_scatter(
                    fracx, [j, jnp.full((_LANES,), c, jnp.int32)], fr)

        @pl.loop(0, bpw)
        def _(bi):
            b = wid * bpw + bi
            boff = b * L

            @pl.loop(0, nchunk)
            def _(r):
                for gg in range(_IDX_CHUNK // _LANES):
                    sl = pl.ds(gg * _LANES, _LANES)
                    idxb[r, sl] = bidx[r, sl] + boff

            for ch in range(nchunk):
                pltpu.async_copy(
                    cf_hbm.at[idxb.at[ch]],
                    rows.at[pl.ds(ch * _IDX_CHUNK, _IDX_CHUNK)],
                    gsem,
                )
            for ch in range(nchunk):
                pltpu.make_async_copy(
                    cf_hbm.at[idxb.at[ch]],
                    rows.at[pl.ds(ch * _IDX_CHUNK, _IDX_CHUNK)],
                    gsem,
                ).wait()

            @pl.loop(0, T, step=8)
            def _(j0):
                for u in range(8):
                    j = j0 + u
                    f = fracx[j, :]
                    p0 = rows[2 * j, pl.ds(0, _LANES)]
                    n0 = rows[2 * j + 1, pl.ds(0, _LANES)]
                    out_v[j, pl.ds(0, _LANES)] = p0 + f * (n0 - p0)
                    p1 = rows[2 * j, pl.ds(_LANES, _LANES)]
                    n1 = rows[2 * j + 1, pl.ds(_LANES, _LANES)]
                    out_v[j, pl.ds(_LANES, _LANES)] = p1 + f * (n1 - p1)

            pltpu.sync_copy(out_v, out_hbm.at[b])

    return k


def kernel(coeffs, t_query):
    B, L, C = coeffs.shape
    T = t_query.shape[0]
    cf = coeffs.reshape(B * L, C)
    return _interp_kernel(B, L, C, T)(cf, t_query)


# speedup vs baseline: 1.1766x; 1.1766x over previous
import dataclasses
import functools

import jax
import jax.numpy as jnp
from jax import lax
from jax.experimental import pallas as pl
from jax.experimental.pallas import tpu as pltpu
from jax.experimental.pallas import tpu_sc as plsc

_LANES = 16
_NWORKERS = 32
_IDX_CHUNK = 128


def _interp_kernel(B, L, C, T):
    bpw = B // _NWORKERS
    nchunk = T // _IDX_CHUNK
    groups = T // _LANES
    gpr = _IDX_CHUNK // _LANES
    mesh = plsc.VectorSubcoreMesh(core_axis_name="c", subcore_axis_name="s")
    cp = pltpu.CompilerParams()
    if "needs_layout_passes" in pltpu.CompilerParams.__dataclass_fields__:
        cp = dataclasses.replace(cp, needs_layout_passes=False)
    if "use_tc_tiling_on_sc" in pltpu.CompilerParams.__dataclass_fields__:
        cp = dataclasses.replace(cp, use_tc_tiling_on_sc=False)

    @functools.partial(
        pl.kernel,
        out_type=jax.ShapeDtypeStruct((B, T, C), jnp.float32),
        mesh=mesh,
        compiler_params=cp,
        scratch_types=[
            pltpu.VMEM((T,), jnp.float32),
            pltpu.VMEM((nchunk, _IDX_CHUNK), jnp.int32),
            pltpu.VMEM((nchunk, _IDX_CHUNK), jnp.int32),
            pltpu.VMEM((nchunk, _IDX_CHUNK), jnp.int32),
            pltpu.VMEM((nchunk, _IDX_CHUNK), jnp.int32),
            pltpu.VMEM((nchunk, _IDX_CHUNK), jnp.int32),
            pltpu.VMEM((T, _LANES), jnp.float32),
            pltpu.VMEM((T, C), jnp.float32),
            pltpu.VMEM((T, C), jnp.float32),
            pltpu.VMEM((T, C), jnp.float32),
            pltpu.VMEM((T, C), jnp.float32),
            pltpu.VMEM((T, C), jnp.float32),
            pltpu.VMEM((T, C), jnp.float32),
            pltpu.SemaphoreType.DMA,
            pltpu.SemaphoreType.DMA,
            pltpu.SemaphoreType.DMA,
            pltpu.SemaphoreType.DMA,
        ],
    )
    def k(cf_hbm, tq_hbm, out_hbm, tq_v, base,
          idxp0, idxn0, idxp1, idxn1, fracx,
          prev0, next0, prev1, next1, outv0, outv1,
          gsem0, gsem1, osem0, osem1):
        wid = lax.axis_index("s") * 2 + lax.axis_index("c")
        b0 = wid * bpw
        pltpu.sync_copy(tq_hbm, tq_v)

        lane = lax.broadcasted_iota(jnp.int32, (_LANES,), 0)

        @pl.loop(0, groups)
        def _(g):
            t = tq_v[pl.ds(g * _LANES, _LANES)]
            ti = jnp.minimum(jnp.maximum(t.astype(jnp.int32), 0), L - 2)
            fr = t - ti.astype(jnp.float32)
            base[g // gpr, pl.ds((g % gpr) * _LANES, _LANES)] = ti
            j = g * _LANES + lane
            for c in range(_LANES):
                plsc.store_scatter(
                    fracx, [j, jnp.full((_LANES,), c, jnp.int32)], fr)

        def build_and_fire(bi, idxp, idxn, prevb, nextb, gsem):
            boff = (b0 + bi) * L
            for r in range(nchunk):
                for gg in range(gpr):
                    sl = pl.ds(gg * _LANES, _LANES)
                    v = base[r, sl] + boff
                    idxp[r, sl] = v
                    idxn[r, sl] = v + 1
            for ch in range(nchunk):
                pltpu.async_copy(
                    cf_hbm.at[idxp.at[ch]],
                    prevb.at[pl.ds(ch * _IDX_CHUNK, _IDX_CHUNK)], gsem)
                pltpu.async_copy(
                    cf_hbm.at[idxn.at[ch]],
                    nextb.at[pl.ds(ch * _IDX_CHUNK, _IDX_CHUNK)], gsem)

        def drain_gather(idxp, idxn, prevb, nextb, gsem):
            for ch in range(nchunk):
                pltpu.make_async_copy(
                    cf_hbm.at[idxp.at[ch]],
                    prevb.at[pl.ds(ch * _IDX_CHUNK, _IDX_CHUNK)], gsem).wait()
                pltpu.make_async_copy(
                    cf_hbm.at[idxn.at[ch]],
                    nextb.at[pl.ds(ch * _IDX_CHUNK, _IDX_CHUNK)], gsem).wait()

        def compute(bi, prevb, nextb, outv, osem):
            @pl.loop(0, T, step=8)
            def _(j0):
                for u in range(8):
                    j = j0 + u
                    f = fracx[j, :]
                    p0 = prevb[j, pl.ds(0, _LANES)]
                    n0 = nextb[j, pl.ds(0, _LANES)]
                    outv[j, pl.ds(0, _LANES)] = p0 + f * (n0 - p0)
                    p1 = prevb[j, pl.ds(_LANES, _LANES)]
                    n1 = nextb[j, pl.ds(_LANES, _LANES)]
                    outv[j, pl.ds(_LANES, _LANES)] = p1 + f * (n1 - p1)
            pltpu.async_copy(outv, out_hbm.at[b0 + bi], osem)

        def wait_out(bi, outv, osem):
            pltpu.make_async_copy(outv, out_hbm.at[b0 + bi], osem).wait()

        build_and_fire(0, idxp0, idxn0, prev0, next0, gsem0)
        build_and_fire(1, idxp1, idxn1, prev1, next1, gsem1)

        @pl.loop(0, bpw, step=2)
        def _(bi):
            drain_gather(idxp0, idxn0, prev0, next0, gsem0)

            @pl.when(bi >= 2)
            def _():
                wait_out(bi - 2, outv0, osem0)

            compute(bi, prev0, next0, outv0, osem0)

            @pl.when(bi + 2 < bpw)
            def _():
                build_and_fire(bi + 2, idxp0, idxn0, prev0, next0, gsem0)

            drain_gather(idxp1, idxn1, prev1, next1, gsem1)

            @pl.when(bi >= 2)
            def _():
                wait_out(bi - 1, outv1, osem1)

            compute(bi + 1, prev1, next1, outv1, osem1)

            @pl.when(bi + 3 < bpw)
            def _():
                build_and_fire(bi + 3, idxp1, idxn1, prev1, next1, gsem1)

        wait_out(bpw - 2, outv0, osem0)
        wait_out(bpw - 1, outv1, osem1)

    return k


def kernel(coeffs, t_query):
    B, L, C = coeffs.shape
    T = t_query.shape[0]
    cf = coeffs.reshape(B * L, C)
    return _interp_kernel(B, L, C, T)(cf, t_query)


# speedup vs baseline: 4.3072x; 3.6606x over previous
import dataclasses
import functools

import jax
import jax.numpy as jnp
from jax import lax
from jax.experimental import pallas as pl
from jax.experimental.pallas import tpu as pltpu
from jax.experimental.pallas import tpu_sc as plsc

_LANES = 16
_NWORKERS = 32


def _interp_kernel(B, L, C, T):
    CT = C // 8
    LT = L // 128
    TT = T // 128
    nslab = B * CT
    spw = nslab // _NWORKERS
    groups = T // _LANES
    mesh = plsc.VectorSubcoreMesh(core_axis_name="c", subcore_axis_name="s")
    cp = pltpu.CompilerParams()
    if "needs_layout_passes" in pltpu.CompilerParams.__dataclass_fields__:
        cp = dataclasses.replace(cp, needs_layout_passes=False)
    if "use_tc_tiling_on_sc" in pltpu.CompilerParams.__dataclass_fields__:
        cp = dataclasses.replace(cp, use_tc_tiling_on_sc=False)

    @functools.partial(
        pl.kernel,
        out_type=jax.ShapeDtypeStruct((nslab, TT, 8, 128), jnp.float32),
        mesh=mesh,
        compiler_params=cp,
        scratch_types=[
            pltpu.VMEM((T,), jnp.float32),
            pltpu.VMEM((T,), jnp.int32),
            pltpu.VMEM((T,), jnp.int32),
            pltpu.VMEM((T,), jnp.int32),
            pltpu.VMEM((T,), jnp.int32),
            pltpu.VMEM((T,), jnp.float32),
            pltpu.VMEM((LT, 8, 128), jnp.float32),
            pltpu.VMEM((LT, 8, 128), jnp.float32),
            pltpu.VMEM((TT, 8, 128), jnp.float32),
            pltpu.VMEM((TT, 8, 128), jnp.float32),
            pltpu.SemaphoreType.DMA,
            pltpu.SemaphoreType.DMA,
            pltpu.SemaphoreType.DMA,
            pltpu.SemaphoreType.DMA,
        ],
    )
    def k(x_hbm, tq_hbm, y_hbm, tq_v, hi_v, lo_v, hi1_v, lo1_v, fr_v,
          knots0, knots1, outv0, outv1, gsem0, gsem1, osem0, osem1):
        wid = lax.axis_index("s") * 2 + lax.axis_index("c")
        k0 = wid * spw
        pltpu.sync_copy(tq_hbm, tq_v)

        @pl.loop(0, groups)
        def _(g):
            sl = pl.ds(g * _LANES, _LANES)
            t = tq_v[sl]
            ti = jnp.minimum(jnp.maximum(t.astype(jnp.int32), 0), L - 2)
            ti1 = ti + 1
            fr_v[sl] = t - ti.astype(jnp.float32)
            hi_v[sl] = lax.shift_right_logical(ti, 7)
            lo_v[sl] = lax.bitwise_and(ti, 127)
            hi1_v[sl] = lax.shift_right_logical(ti1, 7)
            lo1_v[sl] = lax.bitwise_and(ti1, 127)

        civ = [jnp.full((_LANES,), ci, jnp.int32) for ci in range(8)]

        def fire(kk, knots, gsem):
            pltpu.async_copy(x_hbm.at[k0 + kk], knots, gsem)

        def drain(kk, knots, gsem):
            pltpu.make_async_copy(x_hbm.at[k0 + kk], knots, gsem).wait()

        def compute(kk, knots, outv, osem):
            @pl.loop(0, groups)
            def _(g):
                sl = pl.ds(g * _LANES, _LANES)
                hi = hi_v[sl]
                lo = lo_v[sl]
                hi1 = hi1_v[sl]
                lo1 = lo1_v[sl]
                f = fr_v[sl]
                tsl = pl.ds((g % 8) * _LANES, _LANES)
                for ci in range(8):
                    gp = plsc.load_gather(knots, [hi, civ[ci], lo])
                    gn = plsc.load_gather(knots, [hi1, civ[ci], lo1])
                    outv[g // 8, ci, tsl] = gp + f * (gn - gp)
            pltpu.async_copy(outv, y_hbm.at[k0 + kk], osem)

        def wait_out(kk, outv, osem):
            pltpu.make_async_copy(outv, y_hbm.at[k0 + kk], osem).wait()

        fire(0, knots0, gsem0)
        fire(1, knots1, gsem1)

        @pl.loop(0, spw, step=2)
        def _(kk):
            drain(kk, knots0, gsem0)

            @pl.when(kk >= 2)
            def _():
                wait_out(kk - 2, outv0, osem0)

            compute(kk, knots0, outv0, osem0)

            @pl.when(kk + 2 < spw)
            def _():
                fire(kk + 2, knots0, gsem0)

            drain(kk + 1, knots1, gsem1)

            @pl.when(kk >= 2)
            def _():
                wait_out(kk - 1, outv1, osem1)

            compute(kk + 1, knots1, outv1, osem1)

            @pl.when(kk + 3 < spw)
            def _():
                fire(kk + 3, knots1, gsem1)

        wait_out(spw - 2, outv0, osem0)
        wait_out(spw - 1, outv1, osem1)

    return k


def kernel(coeffs, t_query):
    B, L, C = coeffs.shape
    T = t_query.shape[0]
    x = (coeffs.reshape(B, L // 128, 128, C // 8, 8)
         .transpose(0, 3, 1, 4, 2)
         .reshape(B * (C // 8), L // 128, 8, 128))
    y = _interp_kernel(B, L, C, T)(x, t_query)
    out = (y.reshape(B, C // 8, T // 128, 8, 128)
           .transpose(0, 2, 4, 1, 3)
           .reshape(B, T, C))
    return out


# speedup vs baseline: 9.3251x; 2.1650x over previous
import dataclasses
import functools

import jax
import jax.numpy as jnp
from jax import lax
from jax.experimental import pallas as pl
from jax.experimental.pallas import tpu as pltpu
from jax.experimental.pallas import tpu_sc as plsc

_LANES = 16
_NWORKERS = 32


def _interp_kernel(B, L, C, T):
    CT = C // 8
    LT = L // 128
    TT = T // 128
    nslab = B * CT
    spw = nslab // _NWORKERS
    groups = T // _LANES
    mesh = plsc.VectorSubcoreMesh(core_axis_name="c", subcore_axis_name="s")
    cp = pltpu.CompilerParams()
    if "needs_layout_passes" in pltpu.CompilerParams.__dataclass_fields__:
        cp = dataclasses.replace(cp, needs_layout_passes=False)
    if "use_tc_tiling_on_sc" in pltpu.CompilerParams.__dataclass_fields__:
        cp = dataclasses.replace(cp, use_tc_tiling_on_sc=False)

    @functools.partial(
        pl.kernel,
        out_type=jax.ShapeDtypeStruct((nslab, TT, 8, 128), jnp.float32),
        mesh=mesh,
        compiler_params=cp,
        scratch_types=[
            pltpu.VMEM((T,), jnp.float32),
            pltpu.VMEM((T,), jnp.int32),
            pltpu.VMEM((T,), jnp.int32),
            pltpu.VMEM((T,), jnp.int32),
            pltpu.VMEM((T,), jnp.int32),
            pltpu.VMEM((T,), jnp.float32),
            pltpu.VMEM((LT, 8, 128), jnp.float32),
            pltpu.VMEM((LT, 8, 128), jnp.float32),
            pltpu.VMEM((TT, 8, 128), jnp.float32),
            pltpu.VMEM((TT, 8, 128), jnp.float32),
            pltpu.SemaphoreType.DMA,
            pltpu.SemaphoreType.DMA,
            pltpu.SemaphoreType.DMA,
            pltpu.SemaphoreType.DMA,
        ],
    )
    def k(x_hbm, tq_hbm, y_hbm, tq_v, hi_v, lo_v, hi1_v, lo1_v, fr_v,
          knots0, knots1, outv0, outv1, gsem0, gsem1, osem0, osem1):
        wid = lax.axis_index("s") * 2 + lax.axis_index("c")
        k0 = wid * spw
        pltpu.sync_copy(tq_hbm, tq_v)

        @pl.loop(0, groups)
        def _(g):
            sl = pl.ds(g * _LANES, _LANES)
            t = tq_v[sl]
            ti = jnp.minimum(jnp.maximum(t.astype(jnp.int32), 0), L - 2)
            ti1 = ti + 1
            fr_v[sl] = t - ti.astype(jnp.float32)
            hi_v[sl] = lax.shift_right_logical(ti, 7)
            lo_v[sl] = lax.bitwise_and(ti, 127)
            hi1_v[sl] = lax.shift_right_logical(ti1, 7)
            lo1_v[sl] = lax.bitwise_and(ti1, 127)

        civ = [jnp.full((_LANES,), ci, jnp.int32) for ci in range(8)]

        def fire(kk, knots, gsem):
            pltpu.async_copy(x_hbm.at[k0 + kk], knots, gsem)

        def drain(kk, knots, gsem):
            pltpu.make_async_copy(x_hbm.at[k0 + kk], knots, gsem).wait()

        def compute(kk, knots, outv, osem):
            @plsc.parallel_loop(0, groups, unroll=4)
            def _(g):
                sl = pl.ds(g * _LANES, _LANES)
                hi = hi_v[sl]
                lo = lo_v[sl]
                hi1 = hi1_v[sl]
                lo1 = lo1_v[sl]
                f = fr_v[sl]
                tsl = pl.ds((g % 8) * _LANES, _LANES)
                for ci in range(8):
                    gp = plsc.load_gather(knots, [hi, civ[ci], lo])
                    gn = plsc.load_gather(knots, [hi1, civ[ci], lo1])
                    outv[g // 8, ci, tsl] = gp + f * (gn - gp)
            pltpu.async_copy(outv, y_hbm.at[k0 + kk], osem)

        def wait_out(kk, outv, osem):
            pltpu.make_async_copy(outv, y_hbm.at[k0 + kk], osem).wait()

        fire(0, knots0, gsem0)
        fire(1, knots1, gsem1)

        @pl.loop(0, spw, step=2)
        def _(kk):
            drain(kk, knots0, gsem0)

            @pl.when(kk >= 2)
            def _():
                wait_out(kk - 2, outv0, osem0)

            compute(kk, knots0, outv0, osem0)

            @pl.when(kk + 2 < spw)
            def _():
                fire(kk + 2, knots0, gsem0)

            drain(kk + 1, knots1, gsem1)

            @pl.when(kk >= 2)
            def _():
                wait_out(kk - 1, outv1, osem1)

            compute(kk + 1, knots1, outv1, osem1)

            @pl.when(kk + 3 < spw)
            def _():
                fire(kk + 3, knots1, gsem1)

        wait_out(spw - 2, outv0, osem0)
        wait_out(spw - 1, outv1, osem1)

    return k


def kernel(coeffs, t_query):
    B, L, C = coeffs.shape
    T = t_query.shape[0]
    x = (coeffs.reshape(B, L // 128, 128, C // 8, 8)
         .transpose(0, 3, 1, 4, 2)
         .reshape(B * (C // 8), L // 128, 8, 128))
    y = _interp_kernel(B, L, C, T)(x, t_query)
    out = (y.reshape(B, C // 8, T // 128, 8, 128)
           .transpose(0, 2, 4, 1, 3)
           .reshape(B, T, C))
    return out


# speedup vs baseline: 10.9697x; 1.1764x over previous
import dataclasses
import functools

import jax
import jax.numpy as jnp
from jax import lax
from jax.experimental import pallas as pl
from jax.experimental.pallas import tpu as pltpu
from jax.experimental.pallas import tpu_sc as plsc

_LANES = 16
_NWORKERS = 32


def _interp_kernel(B, L, C, T):
    CT = C // 8
    LT = L // 128
    TT = T // 128
    nslab = B * CT
    spw = nslab // _NWORKERS
    groups = T // _LANES
    mesh = plsc.VectorSubcoreMesh(core_axis_name="c", subcore_axis_name="s")
    cp = pltpu.CompilerParams()
    if "needs_layout_passes" in pltpu.CompilerParams.__dataclass_fields__:
        cp = dataclasses.replace(cp, needs_layout_passes=False)
    if "use_tc_tiling_on_sc" in pltpu.CompilerParams.__dataclass_fields__:
        cp = dataclasses.replace(cp, use_tc_tiling_on_sc=False)

    @functools.partial(
        pl.kernel,
        out_type=jax.ShapeDtypeStruct((nslab, TT, 8, 128), jnp.float32),
        mesh=mesh,
        compiler_params=cp,
        scratch_types=[
            pltpu.VMEM((T,), jnp.float32),
            pltpu.VMEM((T,), jnp.int32),
            pltpu.VMEM((T,), jnp.int32),
            pltpu.VMEM((T,), jnp.int32),
            pltpu.VMEM((T,), jnp.int32),
            pltpu.VMEM((T,), jnp.float32),
            pltpu.VMEM((LT, 8, 128), jnp.float32),
            pltpu.VMEM((LT, 8, 128), jnp.float32),
            pltpu.VMEM((LT, 8, 128), jnp.float32),
            pltpu.VMEM((LT, 8, 128), jnp.float32),
            pltpu.VMEM((TT, 8, 128), jnp.float32),
            pltpu.VMEM((TT, 8, 128), jnp.float32),
            pltpu.VMEM((TT, 8, 128), jnp.float32),
            pltpu.VMEM((TT, 8, 128), jnp.float32),
            pltpu.SemaphoreType.DMA,
            pltpu.SemaphoreType.DMA,
            pltpu.SemaphoreType.DMA,
            pltpu.SemaphoreType.DMA,
            pltpu.SemaphoreType.DMA,
            pltpu.SemaphoreType.DMA,
            pltpu.SemaphoreType.DMA,
            pltpu.SemaphoreType.DMA,
        ],
    )
    def k(x_hbm, tq_hbm, y_hbm, tq_v, hi_v, lo_v, hi1_v, lo1_v, fr_v,
          knots0, knots1, knots2, knots3, outv0, outv1, outv2, outv3,
          gsem0, gsem1, gsem2, gsem3, osem0, osem1, osem2, osem3):
        wid = lax.axis_index("s") * 2 + lax.axis_index("c")
        k0 = wid * spw
        pltpu.sync_copy(tq_hbm, tq_v)

        @pl.loop(0, groups)
        def _(g):
            sl = pl.ds(g * _LANES, _LANES)
            t = tq_v[sl]
            ti = jnp.minimum(jnp.maximum(t.astype(jnp.int32), 0), L - 2)
            ti1 = ti + 1
            fr_v[sl] = t - ti.astype(jnp.float32)
            hi_v[sl] = lax.shift_right_logical(ti, 7)
            lo_v[sl] = lax.bitwise_and(ti, 127)
            hi1_v[sl] = lax.shift_right_logical(ti1, 7)
            lo1_v[sl] = lax.bitwise_and(ti1, 127)

        civ = [jnp.full((_LANES,), ci, jnp.int32) for ci in range(8)]

        def fire(kk, knots, gsem):
            pltpu.async_copy(x_hbm.at[k0 + kk], knots, gsem)

        def drain(kk, knots, gsem):
            pltpu.make_async_copy(x_hbm.at[k0 + kk], knots, gsem).wait()

        def compute(kk, knots, outv, osem):
            @plsc.parallel_loop(0, groups, unroll=4)
            def _(g):
                sl = pl.ds(g * _LANES, _LANES)
                hi = hi_v[sl]
                lo = lo_v[sl]
                hi1 = hi1_v[sl]
                lo1 = lo1_v[sl]
                f = fr_v[sl]
                tsl = pl.ds((g % 8) * _LANES, _LANES)
                for ci in range(8):
                    gp = plsc.load_gather(knots, [hi, civ[ci], lo])
                    gn = plsc.load_gather(knots, [hi1, civ[ci], lo1])
                    outv[g // 8, ci, tsl] = gp + f * (gn - gp)
            pltpu.async_copy(outv, y_hbm.at[k0 + kk], osem)

        def wait_out(kk, outv, osem):
            pltpu.make_async_copy(outv, y_hbm.at[k0 + kk], osem).wait()

        slots = ((knots0, outv0, gsem0, osem0),
                 (knots1, outv1, gsem1, osem1),
                 (knots2, outv2, gsem2, osem2),
                 (knots3, outv3, gsem3, osem3))
        nbuf = len(slots)

        for s, (knots, _, gsem, _) in enumerate(slots):
            fire(s, knots, gsem)

        @pl.loop(0, spw, step=nbuf)
        def _(kk):
            for s, (knots, outv, gsem, osem) in enumerate(slots):
                drain(kk + s, knots, gsem)

                @pl.when(kk >= nbuf)
                def _():
                    wait_out(kk + s - nbuf, outv, osem)

                compute(kk + s, knots, outv, osem)

                @pl.when(kk + s + nbuf < spw)
                def _():
                    fire(kk + s + nbuf, knots, gsem)

        for s, (_, outv, _, osem) in enumerate(slots):
            wait_out(spw - nbuf + s, outv, osem)

    return k


def kernel(coeffs, t_query):
    B, L, C = coeffs.shape
    T = t_query.shape[0]
    x = (coeffs.reshape(B, L // 128, 128, C // 8, 8)
         .transpose(0, 3, 1, 4, 2)
         .reshape(B * (C // 8), L // 128, 8, 128))
    y = _interp_kernel(B, L, C, T)(x, t_query)
    out = (y.reshape(B, C // 8, T // 128, 8, 128)
           .transpose(0, 2, 4, 1, 3)
           .reshape(B, T, C))
    return out
